# Initial kernel scaffold; baseline (speedup 1.0000x reference)
import jax
import jax.numpy as jnp
from jax.experimental import pallas as pl


def kernel(m, bases_rad, bases_cir_rad, bases_cir_sph, idx_in, idx_out, idx_out_agg, id_swap, W_dense_ba, W_mlp_rbf, W_down, W_bilinear, W_up_ca, W_up_ac):
    raise NotImplementedError("write your pallas kernel here")


# speedup vs baseline: 1.8399x; 1.8399x over previous
import functools
import math

import jax
import jax.numpy as jnp
from jax import lax
from jax.experimental import pallas as pl
from jax.experimental.pallas import tpu as pltpu
from jax.experimental.pallas import tpu_sc as plsc

N_EDGES = 100000
N_TRIP = 400000
KMAX = 4
NUM_SPH = 7
EMB_IN = 128
TRIP_IN = 32
EMB_CBF = 16
INV_SQRT_2 = 1.0 / math.sqrt(2.0)

E_BLK = 2000
GRID = N_EDGES // E_BLK

_NW = 32
_CH = 128


def _silu(x):
    return x * jax.nn.sigmoid(x)


def _dense_body(m_ref, rad_ref, w1_ref, w2_ref, wd_ref, o_ref):
    x = jnp.dot(m_ref[...], w1_ref[...], preferred_element_type=jnp.float32)
    x = _silu(x)
    r = jnp.dot(rad_ref[...], w2_ref[...], preferred_element_type=jnp.float32)
    x = x * r
    y = jnp.dot(x, wd_ref[...], preferred_element_type=jnp.float32)
    o_ref[...] = _silu(y)


def _tc_dense(m, bases_rad, w1, w2, wd):
    return pl.pallas_call(
        _dense_body,
        grid=(GRID,),
        in_specs=[
            pl.BlockSpec((E_BLK, EMB_IN), lambda i: (i, 0)),
            pl.BlockSpec((E_BLK, 16), lambda i: (i, 0)),
            pl.BlockSpec((EMB_IN, EMB_IN), lambda i: (0, 0)),
            pl.BlockSpec((16, EMB_IN), lambda i: (0, 0)),
            pl.BlockSpec((EMB_IN, TRIP_IN), lambda i: (0, 0)),
        ],
        out_specs=pl.BlockSpec((E_BLK, TRIP_IN), lambda i: (i, 0)),
        out_shape=jax.ShapeDtypeStruct((N_EDGES, TRIP_IN), jnp.float32),
    )(m, bases_rad, w1, w2, wd)


def _trip_body(t_ref, sph_ref, rad_ref, wb_ref, o_ref):
    t = t_ref[...]
    sph = sph_ref[...]
    rad = rad_ref[...]
    ps = []
    for s in range(NUM_SPH):
        acc = None
        for k in range(KMAX):
            term = sph[:, k * NUM_SPH + s:k * NUM_SPH + s + 1] * t[:, k * TRIP_IN:(k + 1) * TRIP_IN]
            acc = term if acc is None else acc + term
        ps.append(acc)
    rbf = []
    for c in range(EMB_CBF):
        acc = None
        for s in range(NUM_SPH):
            term = rad[:, c * NUM_SPH + s:c * NUM_SPH + s + 1] * ps[s]
            acc = term if acc is None else acc + term
        rbf.append(acc)
    big = jnp.concatenate(rbf, axis=1)
    o_ref[...] = jnp.dot(big, wb_ref[...], preferred_element_type=jnp.float32)


EB_TRIP = 800


def _tc_trip(t128, sph28, rad112, wb):
    return pl.pallas_call(
        _trip_body,
        grid=(N_EDGES // EB_TRIP,),
        in_specs=[
            pl.BlockSpec((EB_TRIP, KMAX * TRIP_IN), lambda i: (i, 0)),
            pl.BlockSpec((EB_TRIP, KMAX * NUM_SPH), lambda i: (i, 0)),
            pl.BlockSpec((EB_TRIP, EMB_CBF * NUM_SPH), lambda i: (i, 0)),
            pl.BlockSpec((EMB_CBF * TRIP_IN, TRIP_IN), lambda i: (0, 0)),
        ],
        out_specs=pl.BlockSpec((EB_TRIP, TRIP_IN), lambda i: (i, 0)),
        out_shape=jax.ShapeDtypeStruct((N_EDGES, TRIP_IN), jnp.float32),
    )(t128, sph28, rad112, wb)


def _up_body(x_ref, xp_ref, wca_ref, wac_ref, o_ref):
    a = jnp.dot(x_ref[...], wca_ref[...], preferred_element_type=jnp.float32)
    b = jnp.dot(xp_ref[...], wac_ref[...], preferred_element_type=jnp.float32)
    o_ref[...] = (_silu(a) + _silu(b)) * INV_SQRT_2


def _tc_up(x, xp, wca, wac):
    return pl.pallas_call(
        _up_body,
        grid=(GRID,),
        in_specs=[
            pl.BlockSpec((E_BLK, TRIP_IN), lambda i: (i, 0)),
            pl.BlockSpec((E_BLK, TRIP_IN), lambda i: (i, 0)),
            pl.BlockSpec((TRIP_IN, EMB_IN), lambda i: (0, 0)),
            pl.BlockSpec((TRIP_IN, EMB_IN), lambda i: (0, 0)),
        ],
        out_specs=pl.BlockSpec((E_BLK, EMB_IN), lambda i: (i, 0)),
        out_shape=jax.ShapeDtypeStruct((N_EDGES, EMB_IN), jnp.float32),
    )(x, xp, wca, wac)


@functools.lru_cache(maxsize=None)
def _make_sc_gather(n_idx_rows, table_rows, width):
    mesh = plsc.VectorSubcoreMesh(core_axis_name="c", subcore_axis_name="s")

    @functools.partial(
        pl.kernel,
        out_type=jax.ShapeDtypeStruct((n_idx_rows * _CH, width), jnp.float32),
        mesh=mesh,
        scratch_types=[
            pltpu.VMEM((_CH,), jnp.int32),
            pltpu.VMEM((_CH, width), jnp.float32),
            pltpu.SemaphoreType.DMA,
        ],
        compiler_params=pltpu.CompilerParams(use_tc_tiling_on_sc=False),
    )
    def k(table_hbm, idx_hbm, out_hbm, idx_v, rows_v, sem):
        wid = lax.axis_index("s") * 2 + lax.axis_index("c")
        n_mine = (n_idx_rows - wid + _NW - 1) // _NW

        def body(i, carry):
            row = wid + i * _NW
            pltpu.sync_copy(idx_hbm.at[row], idx_v)
            pltpu.async_copy(table_hbm.at[idx_v], rows_v, sem).wait()
            pltpu.sync_copy(rows_v, out_hbm.at[pl.ds(row * _CH, _CH)])
            return carry

        lax.fori_loop(0, n_mine, body, 0)

    return k


def kernel(m, bases_rad, bases_cir_rad, bases_cir_sph, idx_in, idx_out,
           idx_out_agg, id_swap, W_dense_ba, W_mlp_rbf, W_down, W_bilinear,
           W_up_ca, W_up_ac):
    del idx_out, idx_out_agg

    x_ba = _tc_dense(m, bases_rad, W_dense_ba, W_mlp_rbf, W_down)

    idx2d = idx_in.reshape(N_TRIP // _CH, _CH)
    gather_trip = _make_sc_gather(N_TRIP // _CH, N_EDGES, TRIP_IN)
    t = gather_trip(x_ba, idx2d)
    t128 = t.reshape(N_EDGES, KMAX * TRIP_IN)

    sph28 = bases_cir_sph.reshape(N_EDGES, KMAX * NUM_SPH)
    rad112 = bases_cir_rad.reshape(N_EDGES, EMB_CBF * NUM_SPH)
    x = _tc_trip(t128, sph28, rad112, W_bilinear)

    n_pad_rows = (N_EDGES + _CH - 1) // _CH
    pad = n_pad_rows * _CH - N_EDGES
    idp = jnp.concatenate([id_swap, jnp.zeros((pad,), jnp.int32)]).reshape(
        n_pad_rows, _CH)
    gather_perm = _make_sc_gather(n_pad_rows, N_EDGES, TRIP_IN)
    xp = gather_perm(x, idp)

    return _tc_up(x, xp, W_up_ca, W_up_ac)


# speedup vs baseline: 8.2404x; 4.4786x over previous
import functools
import math

import jax
import jax.numpy as jnp
from jax import lax
from jax.experimental import pallas as pl
from jax.experimental.pallas import tpu as pltpu
from jax.experimental.pallas import tpu_sc as plsc

N_EDGES = 100000
N_TRIP = 400000
KMAX = 4
NUM_SPH = 7
EMB_IN = 128
TRIP_IN = 32
EMB_CBF = 16
INV_SQRT_2 = 1.0 / math.sqrt(2.0)

E_BLK = 2000
GRID = N_EDGES // E_BLK

_NW = 32
_CH = 128


def _silu(x):
    return x * jax.nn.sigmoid(x)


def _dense_body(m_ref, rad_ref, w1_ref, w2_ref, wd_ref, o_ref):
    x = jnp.dot(m_ref[...], w1_ref[...], preferred_element_type=jnp.float32)
    x = _silu(x)
    r = jnp.dot(rad_ref[...], w2_ref[...], preferred_element_type=jnp.float32)
    x = x * r
    y = jnp.dot(x, wd_ref[...], preferred_element_type=jnp.float32)
    o_ref[...] = _silu(y)


def _tc_dense(m, bases_rad, w1, w2, wd):
    return pl.pallas_call(
        _dense_body,
        grid=(GRID,),
        in_specs=[
            pl.BlockSpec((E_BLK, EMB_IN), lambda i: (i, 0)),
            pl.BlockSpec((E_BLK, 16), lambda i: (i, 0)),
            pl.BlockSpec((EMB_IN, EMB_IN), lambda i: (0, 0)),
            pl.BlockSpec((16, EMB_IN), lambda i: (0, 0)),
            pl.BlockSpec((EMB_IN, TRIP_IN), lambda i: (0, 0)),
        ],
        out_specs=pl.BlockSpec((E_BLK, TRIP_IN), lambda i: (i, 0)),
        out_shape=jax.ShapeDtypeStruct((N_EDGES, TRIP_IN), jnp.float32),
    )(m, bases_rad, w1, w2, wd)


EP = 102400
LB = 1024


def _trip_body(t_ref, sph_ref, rad_ref, wbt_ref, o_ref):
    a_ks = []
    for k in range(KMAX):
        acc = None
        for s in range(NUM_SPH):
            term = (rad_ref[s * EMB_CBF:(s + 1) * EMB_CBF, :]
                    * sph_ref[s * KMAX + k:s * KMAX + k + 1, :])
            acc = term if acc is None else acc + term
        a_ks.append(acc)
    rbf_rows = []
    for c in range(EMB_CBF):
        acc = None
        for k in range(KMAX):
            term = a_ks[k][c:c + 1, :] * t_ref[k * TRIP_IN:(k + 1) * TRIP_IN, :]
            acc = term if acc is None else acc + term
        rbf_rows.append(acc)
    big = jnp.concatenate(rbf_rows, axis=0)
    xt = jnp.dot(wbt_ref[...], big, preferred_element_type=jnp.float32)
    o_ref[...] = xt


def _tc_trip(t_t, sph_t, rad_t, wbt):
    return pl.pallas_call(
        _trip_body,
        grid=(EP // LB,),
        in_specs=[
            pl.BlockSpec((KMAX * TRIP_IN, LB), lambda i: (0, i)),
            pl.BlockSpec((KMAX * NUM_SPH, LB), lambda i: (0, i)),
            pl.BlockSpec((NUM_SPH * EMB_CBF, LB), lambda i: (0, i)),
            pl.BlockSpec((TRIP_IN, EMB_CBF * TRIP_IN), lambda i: (0, 0)),
        ],
        out_specs=pl.BlockSpec((TRIP_IN, LB), lambda i: (0, i)),
        out_shape=jax.ShapeDtypeStruct((TRIP_IN, EP), jnp.float32),
    )(t_t, sph_t, rad_t, wbt)


def _up_body(x_ref, xp_ref, wca_ref, wac_ref, o_ref):
    a = jnp.dot(x_ref[...], wca_ref[...], preferred_element_type=jnp.float32)
    b = jnp.dot(xp_ref[...], wac_ref[...], preferred_element_type=jnp.float32)
    o_ref[...] = (_silu(a) + _silu(b)) * INV_SQRT_2


def _tc_up(x, xp, wca, wac):
    return pl.pallas_call(
        _up_body,
        grid=(GRID,),
        in_specs=[
            pl.BlockSpec((E_BLK, TRIP_IN), lambda i: (i, 0)),
            pl.BlockSpec((E_BLK, TRIP_IN), lambda i: (i, 0)),
            pl.BlockSpec((TRIP_IN, EMB_IN), lambda i: (0, 0)),
            pl.BlockSpec((TRIP_IN, EMB_IN), lambda i: (0, 0)),
        ],
        out_specs=pl.BlockSpec((E_BLK, EMB_IN), lambda i: (i, 0)),
        out_shape=jax.ShapeDtypeStruct((N_EDGES, EMB_IN), jnp.float32),
    )(x, xp, wca, wac)


@functools.lru_cache(maxsize=None)
def _make_sc_gather(n_idx_rows, table_rows, width):
    mesh = plsc.VectorSubcoreMesh(core_axis_name="c", subcore_axis_name="s")

    @functools.partial(
        pl.kernel,
        out_type=jax.ShapeDtypeStruct((n_idx_rows * _CH, width), jnp.float32),
        mesh=mesh,
        scratch_types=[
            pltpu.VMEM((_CH,), jnp.int32),
            pltpu.VMEM((_CH, width), jnp.float32),
            pltpu.SemaphoreType.DMA,
        ],
        compiler_params=pltpu.CompilerParams(use_tc_tiling_on_sc=False),
    )
    def k(table_hbm, idx_hbm, out_hbm, idx_v, rows_v, sem):
        wid = lax.axis_index("s") * 2 + lax.axis_index("c")
        n_mine = (n_idx_rows - wid + _NW - 1) // _NW

        def body(i, carry):
            row = wid + i * _NW
            pltpu.sync_copy(idx_hbm.at[row], idx_v)
            pltpu.async_copy(table_hbm.at[idx_v], rows_v, sem).wait()
            pltpu.sync_copy(rows_v, out_hbm.at[pl.ds(row * _CH, _CH)])
            return carry

        lax.fori_loop(0, n_mine, body, 0)

    return k


def kernel(m, bases_rad, bases_cir_rad, bases_cir_sph, idx_in, idx_out,
           idx_out_agg, id_swap, W_dense_ba, W_mlp_rbf, W_down, W_bilinear,
           W_up_ca, W_up_ac):
    del idx_out, idx_out_agg

    x_ba = _tc_dense(m, bases_rad, W_dense_ba, W_mlp_rbf, W_down)

    idx2d = idx_in.reshape(N_TRIP // _CH, _CH)
    gather_trip = _make_sc_gather(N_TRIP // _CH, N_EDGES, TRIP_IN)
    t = gather_trip(x_ba, idx2d)
    t128 = t.reshape(N_EDGES, KMAX * TRIP_IN)

    padw = ((0, 0), (0, EP - N_EDGES))
    t_t = jnp.pad(t128.T, padw)
    sph_t = jnp.pad(
        bases_cir_sph.transpose(2, 1, 0).reshape(NUM_SPH * KMAX, N_EDGES), padw)
    rad_t = jnp.pad(
        bases_cir_rad.transpose(2, 1, 0).reshape(NUM_SPH * EMB_CBF, N_EDGES), padw)
    xt = _tc_trip(t_t, sph_t, rad_t, W_bilinear.T)
    x = xt[:, :N_EDGES].T

    n_pad_rows = (N_EDGES + _CH - 1) // _CH
    pad = n_pad_rows * _CH - N_EDGES
    idp = jnp.concatenate([id_swap, jnp.zeros((pad,), jnp.int32)]).reshape(
        n_pad_rows, _CH)
    gather_perm = _make_sc_gather(n_pad_rows, N_EDGES, TRIP_IN)
    xp = gather_perm(x, idp)

    return _tc_up(x, xp, W_up_ca, W_up_ac)
